# baseline (speedup 1.0000x reference)
import functools

import jax
import jax.numpy as jnp
from jax import lax
from jax.experimental import pallas as pl
from jax.experimental.pallas import tpu as pltpu
from jax.experimental.pallas import tpu_sc as plsc

N = 10000
E = 320000
D = 128
NC = 2
NS = 16
NW = NC * NS
K = 128
EPAD = 323584
EPT = EPAD // NW
CH = EPT // K
RPT = 640
NPAD = RPT * NS

F32 = jnp.float32


def _sc_mesh():
    return plsc.VectorSubcoreMesh(
        core_axis_name="c", subcore_axis_name="s",
        num_cores=NC, num_subcores=NS)


def _deg_body(dst_hbm, ones_hbm, zeros_hbm, out_hbm,
              dst_v0, dst_v1, ones_v, deg_sh, sem0, sem1):
    c = lax.axis_index("c")
    s = lax.axis_index("s")
    wid = c * NS + s
    pltpu.sync_copy(zeros_hbm.at[pl.ds(s * RPT, RPT)],
                    deg_sh.at[pl.ds(s * RPT, RPT)])
    pltpu.sync_copy(ones_hbm, ones_v)
    plsc.subcore_barrier()

    def step(i, dst_v_b, sem_b):
        @pl.when(i >= 2)
        def _():
            pltpu.make_async_copy(ones_v, deg_sh.at[dst_v_b], sem_b).wait()
        base = pl.multiple_of(wid * EPT + i * K, 8)
        pltpu.sync_copy(dst_hbm.at[pl.ds(base, K)], dst_v_b)
        pltpu.async_copy(ones_v, deg_sh.at[dst_v_b], sem_b, add=True)

    def chunk(i, carry):
        @pl.when(i % 2 == 0)
        def _():
            step(i, dst_v0, sem0)

        @pl.when(i % 2 == 1)
        def _():
            step(i, dst_v1, sem1)

        return carry

    lax.fori_loop(0, CH, chunk, 0)
    pltpu.make_async_copy(ones_v, deg_sh.at[dst_v0], sem0).wait()
    pltpu.make_async_copy(ones_v, deg_sh.at[dst_v1], sem1).wait()
    plsc.subcore_barrier()
    pltpu.sync_copy(deg_sh.at[pl.ds(s * RPT, RPT)],
                    out_hbm.at[pl.ds(c * NPAD + s * RPT, RPT)])


def _scatter_body(y_hbm, src_hbm, dst_hbm, zeros_hbm, out_hbm,
                  src_v0, src_v1, dst_v0, dst_v1, rows_v0, rows_v1,
                  acc_sh, sg0, sg1, ss0, ss1):
    c = lax.axis_index("c")
    s = lax.axis_index("s")
    wid = c * NS + s
    pltpu.sync_copy(zeros_hbm.at[pl.ds(s * RPT, RPT)],
                    acc_sh.at[pl.ds(s * RPT, RPT)])
    plsc.subcore_barrier()

    def step(i, src_v, dst_v, rows_v, sg, ss):
        @pl.when(i >= 2)
        def _():
            pltpu.make_async_copy(rows_v, acc_sh.at[dst_v], ss).wait()
        base = pl.multiple_of(wid * EPT + i * K, 8)
        pltpu.sync_copy(src_hbm.at[pl.ds(base, K)], src_v)
        pltpu.sync_copy(dst_hbm.at[pl.ds(base, K)], dst_v)
        pltpu.async_copy(y_hbm.at[src_v], rows_v, sg).wait()
        pltpu.async_copy(rows_v, acc_sh.at[dst_v], ss, add=True)

    def chunk(i, carry):
        @pl.when(i % 2 == 0)
        def _():
            step(i, src_v0, dst_v0, rows_v0, sg0, ss0)

        @pl.when(i % 2 == 1)
        def _():
            step(i, src_v1, dst_v1, rows_v1, sg1, ss1)

        return carry

    lax.fori_loop(0, CH, chunk, 0)
    pltpu.make_async_copy(rows_v0, acc_sh.at[dst_v0], ss0).wait()
    pltpu.make_async_copy(rows_v1, acc_sh.at[dst_v1], ss1).wait()
    plsc.subcore_barrier()
    pltpu.sync_copy(acc_sh.at[pl.ds(s * RPT, RPT)],
                    out_hbm.at[pl.ds(c * NPAD + s * RPT, RPT)])


def _dinv(deg_ref):
    deg = deg_ref[0:N, :] + deg_ref[NPAD:NPAD + N, :] + 1.0
    return lax.rsqrt(deg)


def _mm_body(x_ref, w_ref, o_ref):
    o_ref[...] = jnp.dot(x_ref[...], w_ref[...],
                         preferred_element_type=F32,
                         precision=lax.Precision.HIGHEST)


def _scale_body(deg_ref, xw_ref, o_ref):
    o_ref[...] = xw_ref[...] * _dinv(deg_ref)


def _layer_mm_body(deg_ref, parts_ref, y_ref, b_ref, w_ref, o_ref):
    dinv = _dinv(deg_ref)
    h = dinv * (parts_ref[0:N, :] + parts_ref[NPAD:NPAD + N, :] + y_ref[...]) + b_ref[...]
    o_ref[...] = dinv * jnp.dot(
        h, w_ref[...], preferred_element_type=F32,
        precision=lax.Precision.HIGHEST)


def _final_body(deg_ref, parts_ref, y_ref, b_ref, o_ref):
    o_ref[...] = _dinv(deg_ref) * (
        parts_ref[0:N, :] + parts_ref[NPAD:NPAD + N, :] + y_ref[...]) + b_ref[...]


_tc_mm = pl.pallas_call(
    _mm_body, out_shape=jax.ShapeDtypeStruct((N, D), F32))
_tc_scale = pl.pallas_call(
    _scale_body, out_shape=jax.ShapeDtypeStruct((N, D), F32))
_tc_layer_mm = pl.pallas_call(
    _layer_mm_body, out_shape=jax.ShapeDtypeStruct((N, D), F32))
_tc_final = pl.pallas_call(
    _final_body, out_shape=jax.ShapeDtypeStruct((N, D), F32))


@functools.cache
def _sc_kernels():
    deg_k = pl.kernel(
        _deg_body,
        out_type=jax.ShapeDtypeStruct((NC * NPAD, D), F32),
        mesh=_sc_mesh(),
        scratch_types=[
            pltpu.VMEM((K,), jnp.int32),
            pltpu.VMEM((K,), jnp.int32),
            pltpu.VMEM((K, D), F32),
            pltpu.VMEM_SHARED((NPAD, D), F32),
            pltpu.SemaphoreType.DMA,
            pltpu.SemaphoreType.DMA,
        ],
    )
    scat_k = pl.kernel(
        _scatter_body,
        out_type=jax.ShapeDtypeStruct((NC * NPAD, D), F32),
        mesh=_sc_mesh(),
        scratch_types=[
            pltpu.VMEM((K,), jnp.int32),
            pltpu.VMEM((K,), jnp.int32),
            pltpu.VMEM((K,), jnp.int32),
            pltpu.VMEM((K,), jnp.int32),
            pltpu.VMEM((K, D), F32),
            pltpu.VMEM((K, D), F32),
            pltpu.VMEM_SHARED((NPAD, D), F32),
            pltpu.SemaphoreType.DMA,
            pltpu.SemaphoreType.DMA,
            pltpu.SemaphoreType.DMA,
            pltpu.SemaphoreType.DMA,
        ],
    )
    return deg_k, scat_k


def kernel(features, edge_index, W1, b1, W2, b2):
    _deg_kernel, _scatter_kernel = _sc_kernels()
    ei = edge_index.astype(jnp.int32)
    src = jnp.concatenate([ei[0], jnp.zeros((EPAD - E,), jnp.int32)])
    dst = jnp.concatenate([ei[1], jnp.full((EPAD - E,), N, jnp.int32)])
    zerosD = jnp.zeros((NPAD, D), F32)
    onesK = jnp.ones((K, D), F32)

    xw1 = _tc_mm(features, W1)
    degp = _deg_kernel(dst, onesK, zerosD)
    y1 = _tc_scale(degp, xw1)
    p1 = _scatter_kernel(y1, src, dst, zerosD)
    y2 = _tc_layer_mm(degp, p1, y1, b1.reshape(1, D), W2)
    p2 = _scatter_kernel(y2, src, dst, zerosD)
    return _tc_final(degp, p2, y2, b2.reshape(1, D))


# speedup vs baseline: 1.1859x; 1.1859x over previous
import functools

import jax
import jax.numpy as jnp
from jax import lax
from jax.experimental import pallas as pl
from jax.experimental.pallas import tpu as pltpu
from jax.experimental.pallas import tpu_sc as plsc

N = 10000
E = 320000
D = 128
NC = 2
NS = 16
NW = NC * NS
K = 128
EPAD = 323584
EPT = EPAD // NW
CH = EPT // K
RPT = 640
NPAD = RPT * NS

F32 = jnp.float32


def _sc_mesh():
    return plsc.VectorSubcoreMesh(
        core_axis_name="c", subcore_axis_name="s",
        num_cores=NC, num_subcores=NS)


def _deg_body(dst_hbm, ones_hbm, zeros_hbm, out_hbm,
              dst_v0, dst_v1, ones_v, deg_sh, sem0, sem1):
    c = lax.axis_index("c")
    s = lax.axis_index("s")
    wid = c * NS + s
    pltpu.sync_copy(zeros_hbm.at[pl.ds(s * RPT, RPT)],
                    deg_sh.at[pl.ds(s * RPT, RPT)])
    pltpu.sync_copy(ones_hbm, ones_v)
    plsc.subcore_barrier()

    def step(i, dst_v_b, sem_b):
        @pl.when(i >= 2)
        def _():
            pltpu.make_async_copy(ones_v, deg_sh.at[dst_v_b], sem_b).wait()
        base = pl.multiple_of(wid * EPT + i * K, 8)
        pltpu.sync_copy(dst_hbm.at[pl.ds(base, K)], dst_v_b)
        pltpu.async_copy(ones_v, deg_sh.at[dst_v_b], sem_b, add=True)

    def chunk(i, carry):
        @pl.when(i % 2 == 0)
        def _():
            step(i, dst_v0, sem0)

        @pl.when(i % 2 == 1)
        def _():
            step(i, dst_v1, sem1)

        return carry

    lax.fori_loop(0, CH, chunk, 0)
    pltpu.make_async_copy(ones_v, deg_sh.at[dst_v0], sem0).wait()
    pltpu.make_async_copy(ones_v, deg_sh.at[dst_v1], sem1).wait()
    plsc.subcore_barrier()
    pltpu.sync_copy(deg_sh.at[pl.ds(s * RPT, RPT)],
                    out_hbm.at[pl.ds(c * NPAD + s * RPT, RPT)])


def _scatter_body(y_hbm, src_hbm, dst_hbm, zeros_hbm, out_hbm,
                  src_v0, src_v1, dst_v0, dst_v1, rows_v0, rows_v1,
                  acc_sh, sg0, sg1, ss0, ss1):
    c = lax.axis_index("c")
    s = lax.axis_index("s")
    wid = c * NS + s
    pltpu.sync_copy(zeros_hbm.at[pl.ds(s * RPT, RPT)],
                    acc_sh.at[pl.ds(s * RPT, RPT)])
    plsc.subcore_barrier()

    def step(i, src_v, dst_v, rows_v, sg, ss):
        @pl.when(i >= 2)
        def _():
            pltpu.make_async_copy(rows_v, acc_sh.at[dst_v], ss).wait()
        base = pl.multiple_of(wid * EPT + i * K, 8)
        pltpu.sync_copy(src_hbm.at[pl.ds(base, K)], src_v)
        pltpu.sync_copy(dst_hbm.at[pl.ds(base, K)], dst_v)
        pltpu.async_copy(y_hbm.at[src_v], rows_v, sg).wait()
        pltpu.async_copy(rows_v, acc_sh.at[dst_v], ss, add=True)

    def chunk(i, carry):
        @pl.when(i % 2 == 0)
        def _():
            step(i, src_v0, dst_v0, rows_v0, sg0, ss0)

        @pl.when(i % 2 == 1)
        def _():
            step(i, src_v1, dst_v1, rows_v1, sg1, ss1)

        return carry

    lax.fori_loop(0, CH, chunk, 0)
    pltpu.make_async_copy(rows_v0, acc_sh.at[dst_v0], ss0).wait()
    pltpu.make_async_copy(rows_v1, acc_sh.at[dst_v1], ss1).wait()
    plsc.subcore_barrier()
    pltpu.sync_copy(acc_sh.at[pl.ds(s * RPT, RPT)],
                    out_hbm.at[pl.ds(c * NPAD + s * RPT, RPT)])


def _dinv(deg_ref):
    deg = deg_ref[0:N, 0:1] + deg_ref[NPAD:NPAD + N, 0:1] + 1.0
    return lax.rsqrt(deg)


def _mm_scale_body(deg_ref, x_ref, w_ref, o_ref):
    xw = jnp.dot(x_ref[...], w_ref[...],
                 preferred_element_type=F32, precision=lax.Precision.HIGHEST)
    o_ref[...] = xw * _dinv(deg_ref)


def _layer_mm_body(deg_ref, parts_ref, y_ref, b_ref, w_ref, o_ref):
    dinv = _dinv(deg_ref)
    h = dinv * (parts_ref[0:N, :] + parts_ref[NPAD:NPAD + N, :] + y_ref[...]) + b_ref[...]
    o_ref[...] = dinv * jnp.dot(
        h, w_ref[...], preferred_element_type=F32,
        precision=lax.Precision.HIGHEST)


def _final_body(deg_ref, parts_ref, y_ref, b_ref, o_ref):
    o_ref[...] = _dinv(deg_ref) * (
        parts_ref[0:N, :] + parts_ref[NPAD:NPAD + N, :] + y_ref[...]) + b_ref[...]


_tc_mm_scale = pl.pallas_call(
    _mm_scale_body, out_shape=jax.ShapeDtypeStruct((N, D), F32))
_tc_layer_mm = pl.pallas_call(
    _layer_mm_body, out_shape=jax.ShapeDtypeStruct((N, D), F32))
_tc_final = pl.pallas_call(
    _final_body, out_shape=jax.ShapeDtypeStruct((N, D), F32))


@functools.cache
def _sc_kernels():
    deg_k = pl.kernel(
        _deg_body,
        out_type=jax.ShapeDtypeStruct((NC * NPAD, 16), F32),
        mesh=_sc_mesh(),
        scratch_types=[
            pltpu.VMEM((K,), jnp.int32),
            pltpu.VMEM((K,), jnp.int32),
            pltpu.VMEM((K, 16), F32),
            pltpu.VMEM_SHARED((NPAD, 16), F32),
            pltpu.SemaphoreType.DMA,
            pltpu.SemaphoreType.DMA,
        ],
    )
    scat_k = pl.kernel(
        _scatter_body,
        out_type=jax.ShapeDtypeStruct((NC * NPAD, D), F32),
        mesh=_sc_mesh(),
        scratch_types=[
            pltpu.VMEM((K,), jnp.int32),
            pltpu.VMEM((K,), jnp.int32),
            pltpu.VMEM((K,), jnp.int32),
            pltpu.VMEM((K,), jnp.int32),
            pltpu.VMEM((K, D), F32),
            pltpu.VMEM((K, D), F32),
            pltpu.VMEM_SHARED((NPAD, D), F32),
            pltpu.SemaphoreType.DMA,
            pltpu.SemaphoreType.DMA,
            pltpu.SemaphoreType.DMA,
            pltpu.SemaphoreType.DMA,
        ],
    )
    return deg_k, scat_k


def kernel(features, edge_index, W1, b1, W2, b2):
    _deg_kernel, _scatter_kernel = _sc_kernels()
    ei = edge_index.astype(jnp.int32)
    src = jnp.concatenate([ei[0], jnp.zeros((EPAD - E,), jnp.int32)])
    dst = jnp.concatenate([ei[1], jnp.full((EPAD - E,), N, jnp.int32)])
    zerosD = jnp.zeros((NPAD, D), F32)
    zeros16 = jnp.zeros((NPAD, 16), F32)
    onesK = jnp.ones((K, 16), F32)

    degp = _deg_kernel(dst, onesK, zeros16)
    y1 = _tc_mm_scale(degp, features, W1)
    p1 = _scatter_kernel(y1, src, dst, zerosD)
    y2 = _tc_layer_mm(degp, p1, y1, b1.reshape(1, D), W2)
    p2 = _scatter_kernel(y2, src, dst, zerosD)
    return _tc_final(degp, p2, y2, b2.reshape(1, D))


# speedup vs baseline: 1.2265x; 1.0342x over previous
import functools

import jax
import jax.numpy as jnp
from jax import lax
from jax.experimental import pallas as pl
from jax.experimental.pallas import tpu as pltpu
from jax.experimental.pallas import tpu_sc as plsc

N = 10000
E = 320000
D = 128
NC = 2
NS = 16
NW = NC * NS
K = 128
EPAD = 323584
EPT = EPAD // NW
CH = EPT // K
RPT = 640
NPAD = RPT * NS

F32 = jnp.float32


def _sc_mesh():
    return plsc.VectorSubcoreMesh(
        core_axis_name="c", subcore_axis_name="s",
        num_cores=NC, num_subcores=NS)


def _deg_body(dst_hbm, ones_hbm, zeros_hbm, out_hbm,
              dst_v0, dst_v1, ones_v, deg_sh, sem0, sem1):
    c = lax.axis_index("c")
    s = lax.axis_index("s")
    wid = c * NS + s
    pltpu.sync_copy(zeros_hbm.at[pl.ds(s * RPT, RPT)],
                    deg_sh.at[pl.ds(s * RPT, RPT)])
    pltpu.sync_copy(ones_hbm, ones_v)
    plsc.subcore_barrier()

    def step(i, dst_v_b, sem_b):
        @pl.when(i >= 2)
        def _():
            pltpu.make_async_copy(ones_v, deg_sh.at[dst_v_b], sem_b).wait()
        base = pl.multiple_of(wid * EPT + i * K, 8)
        pltpu.sync_copy(dst_hbm.at[pl.ds(base, K)], dst_v_b)
        pltpu.async_copy(ones_v, deg_sh.at[dst_v_b], sem_b, add=True)

    def chunk(i, carry):
        @pl.when(i % 2 == 0)
        def _():
            step(i, dst_v0, sem0)

        @pl.when(i % 2 == 1)
        def _():
            step(i, dst_v1, sem1)

        return carry

    lax.fori_loop(0, CH, chunk, 0)
    pltpu.make_async_copy(ones_v, deg_sh.at[dst_v0], sem0).wait()
    pltpu.make_async_copy(ones_v, deg_sh.at[dst_v1], sem1).wait()
    plsc.subcore_barrier()
    pltpu.sync_copy(deg_sh.at[pl.ds(s * RPT, RPT)],
                    out_hbm.at[pl.ds(c * NPAD + s * RPT, RPT)])


def _scatter_body(y_hbm, eidx_hbm, zeros_hbm, out_hbm,
                  idx_v0, idx_v1, rows_v0, rows_v1,
                  acc_sh, sg0, sg1, ss0, ss1):
    c = lax.axis_index("c")
    s = lax.axis_index("s")
    wid = c * NS + s
    pltpu.sync_copy(zeros_hbm.at[pl.ds(s * RPT, RPT)],
                    acc_sh.at[pl.ds(s * RPT, RPT)])
    plsc.subcore_barrier()

    def step(i, idx_v, rows_v, sg, ss):
        @pl.when(i >= 2)
        def _():
            pltpu.make_async_copy(rows_v, acc_sh.at[idx_v.at[1]], ss).wait()
        pltpu.sync_copy(eidx_hbm.at[wid * CH + i], idx_v)
        pltpu.async_copy(y_hbm.at[idx_v.at[0]], rows_v, sg).wait()
        pltpu.async_copy(rows_v, acc_sh.at[idx_v.at[1]], ss, add=True)

    def chunk(i, carry):
        @pl.when(i % 2 == 0)
        def _():
            step(i, idx_v0, rows_v0, sg0, ss0)

        @pl.when(i % 2 == 1)
        def _():
            step(i, idx_v1, rows_v1, sg1, ss1)

        return carry

    lax.fori_loop(0, CH, chunk, 0)
    pltpu.make_async_copy(rows_v0, acc_sh.at[idx_v0.at[1]], ss0).wait()
    pltpu.make_async_copy(rows_v1, acc_sh.at[idx_v1.at[1]], ss1).wait()
    plsc.subcore_barrier()
    pltpu.sync_copy(acc_sh.at[pl.ds(s * RPT, RPT)],
                    out_hbm.at[pl.ds(c * NPAD + s * RPT, RPT)])


def _dinv(deg_ref):
    deg = deg_ref[0:N, :] + deg_ref[NPAD:NPAD + N, :] + 1.0
    return lax.rsqrt(deg)


def _mm_scale_body(deg_ref, x_ref, w_ref, o_ref):
    xw = jnp.dot(x_ref[...], w_ref[...],
                 preferred_element_type=F32, precision=lax.Precision.HIGHEST)
    o_ref[...] = xw * _dinv(deg_ref)


def _layer_mm_body(deg_ref, parts_ref, y_ref, b_ref, w_ref, o_ref):
    dinv = _dinv(deg_ref)
    h = dinv * (parts_ref[0:N, :] + parts_ref[NPAD:NPAD + N, :] + y_ref[...]) + b_ref[...]
    o_ref[...] = dinv * jnp.dot(
        h, w_ref[...], preferred_element_type=F32,
        precision=lax.Precision.HIGHEST)


def _final_body(deg_ref, parts_ref, y_ref, b_ref, o_ref):
    o_ref[...] = _dinv(deg_ref) * (
        parts_ref[0:N, :] + parts_ref[NPAD:NPAD + N, :] + y_ref[...]) + b_ref[...]


_tc_mm_scale = pl.pallas_call(
    _mm_scale_body, out_shape=jax.ShapeDtypeStruct((N, D), F32))
_tc_layer_mm = pl.pallas_call(
    _layer_mm_body, out_shape=jax.ShapeDtypeStruct((N, D), F32))
_tc_final = pl.pallas_call(
    _final_body, out_shape=jax.ShapeDtypeStruct((N, D), F32))


@functools.cache
def _sc_kernels():
    deg_k = pl.kernel(
        _deg_body,
        out_type=jax.ShapeDtypeStruct((NC * NPAD, D), F32),
        mesh=_sc_mesh(),
        scratch_types=[
            pltpu.VMEM((K,), jnp.int32),
            pltpu.VMEM((K,), jnp.int32),
            pltpu.VMEM((K, D), F32),
            pltpu.VMEM_SHARED((NPAD, D), F32),
            pltpu.SemaphoreType.DMA,
            pltpu.SemaphoreType.DMA,
        ],
    )
    scat_k = pl.kernel(
        _scatter_body,
        out_type=jax.ShapeDtypeStruct((NC * NPAD, D), F32),
        mesh=_sc_mesh(),
        scratch_types=[
            pltpu.VMEM((2, K), jnp.int32),
            pltpu.VMEM((2, K), jnp.int32),
            pltpu.VMEM((K, D), F32),
            pltpu.VMEM((K, D), F32),
            pltpu.VMEM_SHARED((NPAD, D), F32),
            pltpu.SemaphoreType.DMA,
            pltpu.SemaphoreType.DMA,
            pltpu.SemaphoreType.DMA,
            pltpu.SemaphoreType.DMA,
        ],
    )
    return deg_k, scat_k


def kernel(features, edge_index, W1, b1, W2, b2):
    _deg_kernel, _scatter_kernel = _sc_kernels()
    ei = edge_index.astype(jnp.int32)
    src = jnp.concatenate([ei[0], jnp.zeros((EPAD - E,), jnp.int32)])
    dst = jnp.concatenate([ei[1], jnp.full((EPAD - E,), N, jnp.int32)])
    zerosD = jnp.zeros((NPAD, D), F32)
    onesK = jnp.ones((K, D), F32)

    degp = _deg_kernel(dst, onesK, zerosD)
    y1 = _tc_mm_scale(degp, features, W1)
    eidx = jnp.stack([src, dst], 0).reshape(2, NW * CH, K).transpose(1, 0, 2)
    p1 = _scatter_kernel(y1, eidx, zerosD)
    y2 = _tc_layer_mm(degp, p1, y1, b1.reshape(1, D), W2)
    p2 = _scatter_kernel(y2, eidx, zerosD)
    return _tc_final(degp, p2, y2, b2.reshape(1, D))


# speedup vs baseline: 1.3629x; 1.1112x over previous
import functools

import jax
import jax.numpy as jnp
from jax import lax
from jax.experimental import pallas as pl
from jax.experimental.pallas import tpu as pltpu
from jax.experimental.pallas import tpu_sc as plsc

N = 10000
E = 320000
D = 128
NC = 2
NS = 16
NW = NC * NS
K = 128
EPAD = 323584
EPT = EPAD // NW
CH = EPT // K
CH0 = 103
CH1 = 2 * CH - CH0
RPT = 640
NPAD = RPT * NS

F32 = jnp.float32


def _sc_mesh():
    return plsc.VectorSubcoreMesh(
        core_axis_name="c", subcore_axis_name="s",
        num_cores=NC, num_subcores=NS)


def _deg_body(dst_hbm, ones_hbm, zeros_hbm, out_hbm,
              dst_v0, dst_v1, ones_v, deg_sh, sem0, sem1):
    c = lax.axis_index("c")
    s = lax.axis_index("s")
    wid = c * NS + s
    pltpu.sync_copy(zeros_hbm.at[pl.ds(s * RPT, RPT)],
                    deg_sh.at[pl.ds(s * RPT, RPT)])
    pltpu.sync_copy(ones_hbm, ones_v)
    plsc.subcore_barrier()

    def step(i, dst_v_b, sem_b):
        @pl.when(i >= 2)
        def _():
            pltpu.make_async_copy(ones_v, deg_sh.at[dst_v_b], sem_b).wait()
        base = pl.multiple_of(wid * EPT + i * K, 8)
        pltpu.sync_copy(dst_hbm.at[pl.ds(base, K)], dst_v_b)
        pltpu.async_copy(ones_v, deg_sh.at[dst_v_b], sem_b, add=True)

    def chunk(i, carry):
        @pl.when(i % 2 == 0)
        def _():
            step(i, dst_v0, sem0)

        @pl.when(i % 2 == 1)
        def _():
            step(i, dst_v1, sem1)

        return carry

    lax.fori_loop(0, CH, chunk, 0)
    pltpu.make_async_copy(ones_v, deg_sh.at[dst_v0], sem0).wait()
    pltpu.make_async_copy(ones_v, deg_sh.at[dst_v1], sem1).wait()
    plsc.subcore_barrier()
    pltpu.sync_copy(deg_sh.at[pl.ds(s * RPT, RPT)],
                    out_hbm.at[pl.ds(c * NPAD + s * RPT, RPT)])


def _scatter_body(y_hbm, eidx_hbm, zeros_hbm, out_hbm,
                  idx_v0, idx_v1, rows_v0, rows_v1,
                  acc_sh, sg0, sg1, ss0, ss1):
    c = lax.axis_index("c")
    s = lax.axis_index("s")
    nch = jnp.where(c == 0, CH0, CH1)
    g0 = jnp.where(c == 0, s * CH0, NS * CH0 + s * CH1)
    pltpu.sync_copy(zeros_hbm.at[pl.ds(s * RPT, RPT)],
                    acc_sh.at[pl.ds(s * RPT, RPT)])
    plsc.subcore_barrier()

    def step(i, idx_v, rows_v, sg, ss):
        @pl.when(i >= 2)
        def _():
            pltpu.make_async_copy(rows_v, acc_sh.at[idx_v.at[1]], ss).wait()
        pltpu.sync_copy(eidx_hbm.at[g0 + i], idx_v)
        pltpu.async_copy(y_hbm.at[idx_v.at[0]], rows_v, sg).wait()
        pltpu.async_copy(rows_v, acc_sh.at[idx_v.at[1]], ss, add=True)

    def chunk(i, carry):
        @pl.when(i % 2 == 0)
        def _():
            step(i, idx_v0, rows_v0, sg0, ss0)

        @pl.when(i % 2 == 1)
        def _():
            step(i, idx_v1, rows_v1, sg1, ss1)

        return carry

    lax.fori_loop(0, nch, chunk, 0)
    pltpu.make_async_copy(rows_v0, acc_sh.at[idx_v0.at[1]], ss0).wait()
    pltpu.make_async_copy(rows_v1, acc_sh.at[idx_v1.at[1]], ss1).wait()
    plsc.subcore_barrier()
    pltpu.sync_copy(acc_sh.at[pl.ds(s * RPT, RPT)],
                    out_hbm.at[pl.ds(c * NPAD + s * RPT, RPT)])


def _dinv(deg_ref):
    deg = deg_ref[0:N, :] + deg_ref[NPAD:NPAD + N, :] + 1.0
    return lax.rsqrt(deg)


def _mm_scale_body(deg_ref, x_ref, w_ref, o_ref):
    xw = jnp.dot(x_ref[...], w_ref[...],
                 preferred_element_type=F32, precision=lax.Precision.HIGHEST)
    o_ref[...] = xw * _dinv(deg_ref)


def _layer_mm_body(deg_ref, parts_ref, y_ref, b_ref, w_ref, o_ref):
    dinv = _dinv(deg_ref)
    h = dinv * (parts_ref[0:N, :] + parts_ref[NPAD:NPAD + N, :] + y_ref[...]) + b_ref[...]
    o_ref[...] = dinv * jnp.dot(
        h, w_ref[...], preferred_element_type=F32,
        precision=lax.Precision.HIGHEST)


def _final_body(deg_ref, parts_ref, y_ref, b_ref, o_ref):
    o_ref[...] = _dinv(deg_ref) * (
        parts_ref[0:N, :] + parts_ref[NPAD:NPAD + N, :] + y_ref[...]) + b_ref[...]


_tc_mm_scale = pl.pallas_call(
    _mm_scale_body, out_shape=jax.ShapeDtypeStruct((N, D), F32))
_tc_layer_mm = pl.pallas_call(
    _layer_mm_body, out_shape=jax.ShapeDtypeStruct((N, D), F32))
_tc_final = pl.pallas_call(
    _final_body, out_shape=jax.ShapeDtypeStruct((N, D), F32))


@functools.cache
def _sc_kernels():
    deg_k = pl.kernel(
        _deg_body,
        out_type=jax.ShapeDtypeStruct((NC * NPAD, D), F32),
        mesh=_sc_mesh(),
        scratch_types=[
            pltpu.VMEM((K,), jnp.int32),
            pltpu.VMEM((K,), jnp.int32),
            pltpu.VMEM((K, D), F32),
            pltpu.VMEM_SHARED((NPAD, D), F32),
            pltpu.SemaphoreType.DMA,
            pltpu.SemaphoreType.DMA,
        ],
    )
    scat_k = pl.kernel(
        _scatter_body,
        out_type=jax.ShapeDtypeStruct((NC * NPAD, D), F32),
        mesh=_sc_mesh(),
        scratch_types=[
            pltpu.VMEM((2, K), jnp.int32),
            pltpu.VMEM((2, K), jnp.int32),
            pltpu.VMEM((K, D), F32),
            pltpu.VMEM((K, D), F32),
            pltpu.VMEM_SHARED((NPAD, D), F32),
            pltpu.SemaphoreType.DMA,
            pltpu.SemaphoreType.DMA,
            pltpu.SemaphoreType.DMA,
            pltpu.SemaphoreType.DMA,
        ],
    )
    return deg_k, scat_k


def kernel(features, edge_index, W1, b1, W2, b2):
    _deg_kernel, _scatter_kernel = _sc_kernels()
    ei = edge_index.astype(jnp.int32)
    src = jnp.concatenate([ei[0], jnp.zeros((EPAD - E,), jnp.int32)])
    dst = jnp.concatenate([ei[1], jnp.full((EPAD - E,), N, jnp.int32)])
    zerosD = jnp.zeros((NPAD, D), F32)
    onesK = jnp.ones((K, D), F32)

    degp = _deg_kernel(dst, onesK, zerosD)
    y1 = _tc_mm_scale(degp, features, W1)
    eidx = jnp.stack([src, dst], 0).reshape(2, NW * CH, K).transpose(1, 0, 2)
    p1 = _scatter_kernel(y1, eidx, zerosD)
    y2 = _tc_layer_mm(degp, p1, y1, b1.reshape(1, D), W2)
    p2 = _scatter_kernel(y2, eidx, zerosD)
    return _tc_final(degp, p2, y2, b2.reshape(1, D))


# speedup vs baseline: 1.4356x; 1.0533x over previous
import functools

import jax
import jax.numpy as jnp
from jax import lax
from jax.experimental import pallas as pl
from jax.experimental.pallas import tpu as pltpu
from jax.experimental.pallas import tpu_sc as plsc

N = 10000
E = 320000
D = 128
NC = 2
NS = 16
NW = NC * NS
K = 128
EPAD = 323584
EPT = EPAD // NW
CH = EPT // K
CH0 = 114
CH1 = 2 * CH - CH0
RPT = 640
NPAD = RPT * NS

F32 = jnp.float32


def _sc_mesh():
    return plsc.VectorSubcoreMesh(
        core_axis_name="c", subcore_axis_name="s",
        num_cores=NC, num_subcores=NS)


def _deg_body(dst_hbm, ones_hbm, zeros_hbm, out_hbm,
              dst_v0, dst_v1, ones_v, deg_sh, sem0, sem1):
    c = lax.axis_index("c")
    s = lax.axis_index("s")
    wid = c * NS + s
    pltpu.sync_copy(zeros_hbm.at[pl.ds(s * RPT, RPT)],
                    deg_sh.at[pl.ds(s * RPT, RPT)])
    pltpu.sync_copy(ones_hbm, ones_v)
    plsc.subcore_barrier()

    def step(i, dst_v_b, sem_b):
        @pl.when(i >= 2)
        def _():
            pltpu.make_async_copy(ones_v, deg_sh.at[dst_v_b], sem_b).wait()
        base = pl.multiple_of(wid * EPT + i * K, 8)
        pltpu.sync_copy(dst_hbm.at[pl.ds(base, K)], dst_v_b)
        pltpu.async_copy(ones_v, deg_sh.at[dst_v_b], sem_b, add=True)

    def chunk(i, carry):
        @pl.when(i % 2 == 0)
        def _():
            step(i, dst_v0, sem0)

        @pl.when(i % 2 == 1)
        def _():
            step(i, dst_v1, sem1)

        return carry

    lax.fori_loop(0, CH, chunk, 0)
    pltpu.make_async_copy(ones_v, deg_sh.at[dst_v0], sem0).wait()
    pltpu.make_async_copy(ones_v, deg_sh.at[dst_v1], sem1).wait()
    plsc.subcore_barrier()
    pltpu.sync_copy(deg_sh.at[pl.ds(s * RPT, RPT)],
                    out_hbm.at[pl.ds(c * NPAD + s * RPT, RPT)])


def _scatter_body(y_hbm, eidx_hbm, zeros_hbm, out_hbm,
                  idx_v0, idx_v1, rows_v0, rows_v1,
                  acc_sh, sg0, sg1, ss0, ss1):
    c = lax.axis_index("c")
    s = lax.axis_index("s")
    nch = jnp.where(c == 0, CH0, CH1)
    g0 = jnp.where(c == 0, s * CH0, NS * CH0 + s * CH1)
    pltpu.sync_copy(zeros_hbm.at[pl.ds(s * RPT, RPT)],
                    acc_sh.at[pl.ds(s * RPT, RPT)])
    plsc.subcore_barrier()

    def step(i, idx_v, rows_v, sg, ss):
        @pl.when(i >= 2)
        def _():
            pltpu.make_async_copy(rows_v, acc_sh.at[idx_v.at[1]], ss).wait()
        pltpu.sync_copy(eidx_hbm.at[g0 + i], idx_v)
        pltpu.async_copy(y_hbm.at[idx_v.at[0]], rows_v, sg).wait()
        pltpu.async_copy(rows_v, acc_sh.at[idx_v.at[1]], ss, add=True)

    def chunk(i, carry):
        @pl.when(i % 2 == 0)
        def _():
            step(i, idx_v0, rows_v0, sg0, ss0)

        @pl.when(i % 2 == 1)
        def _():
            step(i, idx_v1, rows_v1, sg1, ss1)

        return carry

    lax.fori_loop(0, nch, chunk, 0)
    pltpu.make_async_copy(rows_v0, acc_sh.at[idx_v0.at[1]], ss0).wait()
    pltpu.make_async_copy(rows_v1, acc_sh.at[idx_v1.at[1]], ss1).wait()
    plsc.subcore_barrier()
    pltpu.sync_copy(acc_sh.at[pl.ds(s * RPT, RPT)],
                    out_hbm.at[pl.ds(c * NPAD + s * RPT, RPT)])


def _dinv(deg_ref):
    deg = deg_ref[0:N, :] + deg_ref[NPAD:NPAD + N, :] + 1.0
    return lax.rsqrt(deg)


def _mm_scale_body(deg_ref, x_ref, w_ref, o_ref):
    xw = jnp.dot(x_ref[...], w_ref[...],
                 preferred_element_type=F32, precision=lax.Precision.HIGHEST)
    o_ref[...] = xw * _dinv(deg_ref)


def _layer_mm_body(deg_ref, parts_ref, y_ref, b_ref, w_ref, o_ref):
    dinv = _dinv(deg_ref)
    h = dinv * (parts_ref[0:N, :] + parts_ref[NPAD:NPAD + N, :] + y_ref[...]) + b_ref[...]
    o_ref[...] = dinv * jnp.dot(
        h, w_ref[...], preferred_element_type=F32,
        precision=lax.Precision.HIGHEST)


def _final_body(deg_ref, parts_ref, y_ref, b_ref, o_ref):
    o_ref[...] = _dinv(deg_ref) * (
        parts_ref[0:N, :] + parts_ref[NPAD:NPAD + N, :] + y_ref[...]) + b_ref[...]


_tc_mm_scale = pl.pallas_call(
    _mm_scale_body, out_shape=jax.ShapeDtypeStruct((N, D), F32))
_tc_layer_mm = pl.pallas_call(
    _layer_mm_body, out_shape=jax.ShapeDtypeStruct((N, D), F32))
_tc_final = pl.pallas_call(
    _final_body, out_shape=jax.ShapeDtypeStruct((N, D), F32))


@functools.cache
def _sc_kernels():
    deg_k = pl.kernel(
        _deg_body,
        out_type=jax.ShapeDtypeStruct((NC * NPAD, D), F32),
        mesh=_sc_mesh(),
        scratch_types=[
            pltpu.VMEM((K,), jnp.int32),
            pltpu.VMEM((K,), jnp.int32),
            pltpu.VMEM((K, D), F32),
            pltpu.VMEM_SHARED((NPAD, D), F32),
            pltpu.SemaphoreType.DMA,
            pltpu.SemaphoreType.DMA,
        ],
    )
    scat_k = pl.kernel(
        _scatter_body,
        out_type=jax.ShapeDtypeStruct((NC * NPAD, D), F32),
        mesh=_sc_mesh(),
        scratch_types=[
            pltpu.VMEM((2, K), jnp.int32),
            pltpu.VMEM((2, K), jnp.int32),
            pltpu.VMEM((K, D), F32),
            pltpu.VMEM((K, D), F32),
            pltpu.VMEM_SHARED((NPAD, D), F32),
            pltpu.SemaphoreType.DMA,
            pltpu.SemaphoreType.DMA,
            pltpu.SemaphoreType.DMA,
            pltpu.SemaphoreType.DMA,
        ],
    )
    return deg_k, scat_k


def kernel(features, edge_index, W1, b1, W2, b2):
    _deg_kernel, _scatter_kernel = _sc_kernels()
    ei = edge_index.astype(jnp.int32)
    src = jnp.concatenate([ei[0], jnp.zeros((EPAD - E,), jnp.int32)])
    dst = jnp.concatenate([ei[1], jnp.full((EPAD - E,), N, jnp.int32)])
    zerosD = jnp.zeros((NPAD, D), F32)
    onesK = jnp.ones((K, D), F32)

    degp = _deg_kernel(dst, onesK, zerosD)
    y1 = _tc_mm_scale(degp, features, W1)
    eidx = jnp.stack([src, dst], 0).reshape(2, NW * CH, K).transpose(1, 0, 2)
    p1 = _scatter_kernel(y1, eidx, zerosD)
    y2 = _tc_layer_mm(degp, p1, y1, b1.reshape(1, D), W2)
    p2 = _scatter_kernel(y2, eidx, zerosD)
    return _tc_final(degp, p2, y2, b2.reshape(1, D))
